# baseline (speedup 1.0000x reference)
import functools

import jax
import jax.numpy as jnp
from jax import lax
from jax.experimental import pallas as pl
from jax.experimental.pallas import tpu as pltpu
from jax.experimental.pallas import tpu_sc as plsc

_MAXP = 15000
_D = 64
_B = 16384
_H = 200
_N = _B * _H
_NC = 2
_NS = 16
_L = 16
_NW = _NC * _NS
_PER_W = _N // _NW
_BLK = 128
_NB = _PER_W // _BLK
_NGP = _NB // 2


def _make_kernel():
    mesh = plsc.VectorSubcoreMesh(core_axis_name="c", subcore_axis_name="s")

    @functools.partial(
        pl.kernel,
        mesh=mesh,
        out_type=jax.ShapeDtypeStruct((_H, 8, _B // _BLK, 8 * _BLK),
                                      jnp.float32),
        scratch_types=[
            pltpu.VMEM((_BLK,), jnp.int32),
            pltpu.VMEM((_BLK,), jnp.int32),
            pltpu.VMEM((_BLK, _D), jnp.float32),
            pltpu.VMEM((_BLK, _D), jnp.float32),
            pltpu.VMEM((8 * 8 * _BLK,), jnp.float32),
            pltpu.VMEM((8 * 8 * _BLK,), jnp.float32),
            pltpu.SemaphoreType.DMA,
            pltpu.SemaphoreType.DMA,
            pltpu.SemaphoreType.DMA,
            pltpu.SemaphoreType.DMA,
            pltpu.SemaphoreType.DMA,
        ],
        compiler_params=pltpu.CompilerParams(
            use_tc_tiling_on_sc=False, needs_layout_passes=False),
    )
    def emb(ids_hbm, pe_hbm, out_hbm, idx0, idx1, rows0, rows1,
            tiles0, tiles1, isem, gsem0, gsem1, wsem0, wsem1):
        wid = lax.axis_index("s") * _NC + lax.axis_index("c")
        base = wid * _PER_W
        idxs = (idx0, idx1)
        rows = (rows0, rows1)
        tiles = (tiles0, tiles1)
        gsems = (gsem0, gsem1)
        wsems = (wsem0, wsem1)

        iota = lax.iota(jnp.int32, _L)
        bvec = [(iota + k) % _L for k in range(_L)]
        cvec = [iota + c0 for c0 in range(0, _D, _L)]
        svec = [((c0 + iota) // 8) * (8 * _BLK) + ((c0 + iota) % 8) * _BLK
                for c0 in range(0, _D, _L)]

        def idx_load(i, b):
            return pltpu.make_async_copy(
                ids_hbm.at[pl.ds(base + i * _BLK, _BLK)], idxs[b], isem)

        def gather(b):
            return pltpu.make_async_copy(
                pe_hbm.at[idxs[b]], rows[b], gsems[b])

        def tile_writes(i, b):
            n0 = base + i * _BLK
            h = n0 >> 14
            bhi = (n0 >> 7) & 127
            return [
                pltpu.make_async_copy(
                    tiles[b].at[pl.ds(chi * 8 * _BLK, 8 * _BLK)],
                    out_hbm.at[h, chi, bhi], wsems[b])
                for chi in range(8)
            ]

        def clamp(b):
            for k in range(8):
                sl = pl.ds(k * _L, _L)
                idxs[b][sl] = jnp.minimum(idxs[b][sl], _MAXP - 1)

        def transpose(b):
            @plsc.parallel_loop(0, _BLK, step=_L)
            def tb(b0, b=b):
                for k in range(_L):
                    rvec = bvec[k] + b0
                    for cg in range(_D // _L):
                        v = plsc.load_gather(rows[b], [rvec, cvec[cg]])
                        plsc.store_scatter(tiles[b], [svec[cg] + rvec], v)

        idx_load(0, 0).start()
        idx_load(0, 0).wait()
        clamp(0)
        gather(0).start()
        idx_load(1, 1).start()

        def body(g, carry):
            for b in range(2):
                i = 2 * g + b
                nb = 1 - b

                def _adv():
                    idx_load(i + 1, nb).wait()
                    clamp(nb)
                    gather(nb).start()
                    return None
                def _prefetch(i=i, b=b):
                    idx_load(i + 2, b).start()
                    return None
                def _drain_writes(i=i, b=b):
                    for c in tile_writes(i - 2, b):
                        c.wait()
                    return None
                if b == 0:
                    _adv()
                else:
                    pl.when(g < _NGP - 1)(_adv)
                gather(b).wait()
                pl.when(g < _NGP - 1)(_prefetch)
                pl.when(g > 0)(_drain_writes)
                transpose(b)
                for c in tile_writes(i, b):
                    c.start()
            return carry

        lax.fori_loop(0, _NGP, body, 0)
        for c in tile_writes(_NB - 2, 0):
            c.wait()
        for c in tile_writes(_NB - 1, 1):
            c.wait()

    return emb


_emb = _make_kernel()


def kernel(position_ids, pe):
    ids_t = jnp.swapaxes(position_ids, 0, 1).reshape(-1)
    out4 = _emb(ids_t, pe)
    out5 = out4.reshape(_H, 8, _B // _BLK, 8, _BLK)
    return out5.transpose(2, 4, 0, 1, 3).reshape(_B, _H, _D)


# speedup vs baseline: 1.7511x; 1.7511x over previous
import functools

import jax
import jax.numpy as jnp
from jax import lax
from jax.experimental import pallas as pl
from jax.experimental.pallas import tpu as pltpu
from jax.experimental.pallas import tpu_sc as plsc

_MAXP = 15000
_D = 64
_B = 16384
_H = 200
_N = _B * _H
_NC = 2
_NS = 16
_L = 16
_NW = _NC * _NS
_PER_W = _N // _NW
_BLK = 128
_NB = _PER_W // _BLK
_NGP = _NB // 2


def _make_kernel():
    mesh = plsc.VectorSubcoreMesh(core_axis_name="c", subcore_axis_name="s")

    @functools.partial(
        pl.kernel,
        mesh=mesh,
        out_type=jax.ShapeDtypeStruct((_H, 8, _B // _BLK, 8 * _BLK),
                                      jnp.float32),
        scratch_types=[
            pltpu.VMEM((_BLK,), jnp.int32),
            pltpu.VMEM((_BLK,), jnp.int32),
            pltpu.VMEM((_BLK, _D), jnp.float32),
            pltpu.VMEM((_BLK, _D), jnp.float32),
            pltpu.VMEM((8 * 8 * _BLK,), jnp.float32),
            pltpu.VMEM((8 * 8 * _BLK,), jnp.float32),
            pltpu.SemaphoreType.DMA,
            pltpu.SemaphoreType.DMA,
            pltpu.SemaphoreType.DMA,
            pltpu.SemaphoreType.DMA,
            pltpu.SemaphoreType.DMA,
        ],
        compiler_params=pltpu.CompilerParams(
            use_tc_tiling_on_sc=False, needs_layout_passes=False),
    )
    def emb(ids_hbm, pe_hbm, out_hbm, idx0, idx1, rows0, rows1,
            tiles0, tiles1, isem, gsem0, gsem1, wsem0, wsem1):
        wid = lax.axis_index("s") * _NC + lax.axis_index("c")
        base = wid * _PER_W
        idxs = (idx0, idx1)
        rows = (rows0, rows1)
        tiles = (tiles0, tiles1)
        gsems = (gsem0, gsem1)
        wsems = (wsem0, wsem1)

        iota = lax.iota(jnp.int32, _L)
        bvec = [(iota + k) % _L for k in range(_L)]
        cvec = [iota + c0 for c0 in range(0, _D, _L)]
        svec = [((c0 + iota) // 8) * (8 * _BLK) + ((c0 + iota) % 8) * _BLK
                for c0 in range(0, _D, _L)]

        def idx_load(i, b):
            return pltpu.make_async_copy(
                ids_hbm.at[pl.ds(base + i * _BLK, _BLK)], idxs[b], isem)

        def gather(b):
            return pltpu.make_async_copy(
                pe_hbm.at[idxs[b]], rows[b], gsems[b])

        def tile_writes(i, b):
            n0 = base + i * _BLK
            h = n0 >> 14
            bhi = (n0 >> 7) & 127
            return [
                pltpu.make_async_copy(
                    tiles[b].at[pl.ds(chi * 8 * _BLK, 8 * _BLK)],
                    out_hbm.at[h, chi, bhi], wsems[b])
                for chi in range(8)
            ]

        def clamp(b):
            for k in range(8):
                sl = pl.ds(k * _L, _L)
                idxs[b][sl] = jnp.minimum(idxs[b][sl], _MAXP - 1)

        def transpose(b):
            for cg in range(_D // _L):
                @plsc.parallel_loop(0, _BLK, step=_L, unroll=4)
                def tb(b0, cg=cg, b=b):
                    for k in range(_L):
                        rvec = bvec[k] + b0
                        v = plsc.load_gather(rows[b], [rvec, cvec[cg]])
                        plsc.store_scatter(tiles[b], [svec[cg] + rvec], v)

        idx_load(0, 0).start()
        idx_load(0, 0).wait()
        clamp(0)
        gather(0).start()
        idx_load(1, 1).start()

        def body(g, carry):
            for b in range(2):
                i = 2 * g + b
                nb = 1 - b

                def _adv():
                    idx_load(i + 1, nb).wait()
                    clamp(nb)
                    gather(nb).start()
                    return None
                def _prefetch(i=i, b=b):
                    idx_load(i + 2, b).start()
                    return None
                def _drain_writes(i=i, b=b):
                    for c in tile_writes(i - 2, b):
                        c.wait()
                    return None
                if b == 0:
                    _adv()
                else:
                    pl.when(g < _NGP - 1)(_adv)
                gather(b).wait()
                pl.when(g < _NGP - 1)(_prefetch)
                pl.when(g > 0)(_drain_writes)
                transpose(b)
                for c in tile_writes(i, b):
                    c.start()
            return carry

        lax.fori_loop(0, _NGP, body, 0)
        for c in tile_writes(_NB - 2, 0):
            c.wait()
        for c in tile_writes(_NB - 1, 1):
            c.wait()

    return emb


_emb = _make_kernel()


def kernel(position_ids, pe):
    ids_t = jnp.swapaxes(position_ids, 0, 1).reshape(-1)
    out4 = _emb(ids_t, pe)
    out5 = out4.reshape(_H, 8, _B // _BLK, 8, _BLK)
    return out5.transpose(2, 4, 0, 1, 3).reshape(_B, _H, _D)


# speedup vs baseline: 1.7572x; 1.0035x over previous
import functools

import jax
import jax.numpy as jnp
from jax import lax
from jax.experimental import pallas as pl
from jax.experimental.pallas import tpu as pltpu
from jax.experimental.pallas import tpu_sc as plsc

_MAXP = 15000
_D = 64
_B = 16384
_H = 200
_N = _B * _H
_NC = 2
_NS = 16
_L = 16
_NW = _NC * _NS
_PER_W = _N // _NW
_BLK = 128
_NB = _PER_W // _BLK
_NGP = _NB // 2


def _make_kernel():
    mesh = plsc.VectorSubcoreMesh(core_axis_name="c", subcore_axis_name="s")

    @functools.partial(
        pl.kernel,
        mesh=mesh,
        out_type=jax.ShapeDtypeStruct((_H, 8, _B // _BLK, 8 * _BLK),
                                      jnp.float32),
        scratch_types=[
            pltpu.VMEM((_BLK,), jnp.int32),
            pltpu.VMEM((_BLK,), jnp.int32),
            pltpu.VMEM((_BLK, _D), jnp.float32),
            pltpu.VMEM((_BLK, _D), jnp.float32),
            pltpu.VMEM((8 * 8 * _BLK,), jnp.float32),
            pltpu.VMEM((8 * 8 * _BLK,), jnp.float32),
            pltpu.SemaphoreType.DMA,
            pltpu.SemaphoreType.DMA,
            pltpu.SemaphoreType.DMA,
            pltpu.SemaphoreType.DMA,
            pltpu.SemaphoreType.DMA,
        ],
        compiler_params=pltpu.CompilerParams(
            use_tc_tiling_on_sc=False, needs_layout_passes=False),
    )
    def emb(ids_hbm, pe_hbm, out_hbm, idx0, idx1, rows0, rows1,
            tiles0, tiles1, isem, gsem0, gsem1, wsem0, wsem1):
        wid = lax.axis_index("s") * _NC + lax.axis_index("c")
        base = wid * _PER_W
        idxs = (idx0, idx1)
        rows = (rows0, rows1)
        tiles = (tiles0, tiles1)
        gsems = (gsem0, gsem1)
        wsems = (wsem0, wsem1)

        iota = lax.iota(jnp.int32, _L)
        bvec = [(iota + k) % _L for k in range(_L)]
        cvec = [iota + c0 for c0 in range(0, _D, _L)]
        svec = [((c0 + iota) // 8) * (8 * _BLK) + ((c0 + iota) % 8) * _BLK
                for c0 in range(0, _D, _L)]

        def idx_load(i, b):
            return pltpu.make_async_copy(
                ids_hbm.at[pl.ds(base + i * _BLK, _BLK)], idxs[b], isem)

        def gather(b):
            return pltpu.make_async_copy(
                pe_hbm.at[idxs[b]], rows[b], gsems[b])

        def tile_writes(i, b):
            n0 = base + i * _BLK
            h = n0 >> 14
            bhi = (n0 >> 7) & 127
            return [
                pltpu.make_async_copy(
                    tiles[b].at[pl.ds(chi * 8 * _BLK, 8 * _BLK)],
                    out_hbm.at[h, chi, bhi], wsems[b])
                for chi in range(8)
            ]

        def clamp(b):
            for k in range(8):
                sl = pl.ds(k * _L, _L)
                idxs[b][sl] = jnp.minimum(idxs[b][sl], _MAXP - 1)

        def transpose_and_write(i, b):
            wr = tile_writes(i, b)
            for cg in range(_D // _L):
                @plsc.parallel_loop(0, _BLK, step=_L, unroll=4)
                def tb(b0, cg=cg, b=b):
                    for k in range(_L):
                        rvec = bvec[k] + b0
                        v = plsc.load_gather(rows[b], [rvec, cvec[cg]])
                        plsc.store_scatter(tiles[b], [svec[cg] + rvec], v)
                wr[2 * cg].start()
                wr[2 * cg + 1].start()

        idx_load(0, 0).start()
        idx_load(0, 0).wait()
        clamp(0)
        gather(0).start()
        idx_load(1, 1).start()

        def body(g, carry):
            for b in range(2):
                i = 2 * g + b
                nb = 1 - b

                def _adv():
                    idx_load(i + 1, nb).wait()
                    clamp(nb)
                    gather(nb).start()
                    return None
                def _prefetch(i=i, b=b):
                    idx_load(i + 2, b).start()
                    return None
                def _drain_writes(i=i, b=b):
                    for c in tile_writes(i - 2, b):
                        c.wait()
                    return None
                if b == 0:
                    _adv()
                else:
                    pl.when(g < _NGP - 1)(_adv)
                gather(b).wait()
                pl.when(g < _NGP - 1)(_prefetch)
                pl.when(g > 0)(_drain_writes)
                transpose_and_write(i, b)
            return carry

        lax.fori_loop(0, _NGP, body, 0)
        for c in tile_writes(_NB - 2, 0):
            c.wait()
        for c in tile_writes(_NB - 1, 1):
            c.wait()

    return emb


_emb = _make_kernel()


def kernel(position_ids, pe):
    ids_t = jnp.swapaxes(position_ids, 0, 1).reshape(-1)
    out4 = _emb(ids_t, pe)
    out5 = out4.reshape(_H, 8, _B // _BLK, 8, _BLK)
    return out5.transpose(2, 4, 0, 1, 3).reshape(_B, _H, _D)
